# baseline (speedup 1.0000x reference)
import functools

import jax
import jax.numpy as jnp
from jax import lax
from jax.experimental import pallas as pl
from jax.experimental.pallas import tpu as pltpu
from jax.experimental.pallas import tpu_sc as plsc

HOUSE_DIM = 64
TIME_DIM = 64
NC, NS, L = 2, 16, 16
NW = NC * NS
TCW = 128
MLP_BLK = 2048


def _bcast(x, dtype=jnp.int32):
    return jnp.full((L,), x, dtype)


def _extract_phase(ids_hbm, tt_hbm, out_hbm, ids_v, mid_v, mpos_v, buf2,
                   stag2, sem, sem2, lo, hi, B):
    pltpu.sync_copy(ids_hbm, ids_v)

    def scan_step(k, n):
        v = ids_v[pl.ds(k * L, L)]
        m = (v >= _bcast(lo)) & (v < _bcast(hi))
        pos = lax.iota(jnp.int32, L) + _bcast(k * L)
        mi = m.astype(jnp.int32)
        cum = plsc.cumsum(mi)
        tgt = jnp.where(m, _bcast(n - 1) + cum, _bcast(B + L))
        plsc.store_scatter(mid_v, [tgt], v)
        plsc.store_scatter(mpos_v, [tgt], pos)
        return n + jnp.sum(mi)

    n = lax.fori_loop(0, B // L, scan_step, jnp.int32(0))

    nch = (hi - lo + TCW - 1) // TCW
    ngrp = (n + L - 1) // L

    def start_chunk(c, r):
        off = lo + c * TCW
        return pltpu.async_copy(tt_hbm.at[:, pl.ds(off, TCW)], buf2.at[r],
                                sem)

    @pl.when(nch > 0)
    def _prime():
        start_chunk(0, 0)

    def chunk_step(c, it):
        r = c % 2
        off = lo + c * TCW
        pltpu.make_async_copy(tt_hbm.at[:, pl.ds(off, TCW)], buf2.at[r],
                              sem).wait()

        @pl.when(c + 1 < nch)
        def _prefetch():
            start_chunk(c + 1, (c + 1) % 2)

        def grp_step(g, it):
            vid = mid_v[pl.ds(g * L, L)]
            vpos = mpos_v[pl.ds(g * L, L)]
            valid = (lax.iota(jnp.int32, L) + _bcast(g * L)) < _bcast(n)
            m = valid & (vid >= _bcast(off)) & (vid < _bcast(off + TCW))
            nact = jnp.sum(m.astype(jnp.int32))

            @pl.when(nact > 0)
            def _work():
                s = it % 2
                @pl.when(it >= 2)
                def _drain():
                    pltpu.make_async_copy(
                        stag2.at[s], out_hbm.at[lax.iota(jnp.int32, L)],
                        sem2).wait()

                col = jnp.where(m, vid - _bcast(off), _bcast(0))
                for f in range(HOUSE_DIM):
                    vals = plsc.load_gather(buf2.at[r], [_bcast(f), col])
                    plsc.store_scatter(
                        stag2.at[s],
                        [lax.iota(jnp.int32, L), _bcast(f)], vals)
                tgt = jnp.where(m, vpos, _bcast(B) + lax.iota(jnp.int32, L))
                pltpu.async_copy(stag2.at[s], out_hbm.at[tgt], sem2)

            return it + jnp.where(nact > 0, 1, 0)

        return lax.fori_loop(0, ngrp, grp_step, it)

    it = lax.fori_loop(0, nch, chunk_step, jnp.int32(0))

    @pl.when(it >= 2)
    def _d2():
        pltpu.make_async_copy(stag2.at[0], out_hbm.at[lax.iota(jnp.int32, L)],
                              sem2).wait()

    @pl.when(it >= 1)
    def _d1():
        pltpu.make_async_copy(stag2.at[0], out_hbm.at[lax.iota(jnp.int32, L)],
                              sem2).wait()


def _gather_body(B, VH, VT, h_tc_pw, t_tc_pw,
                 hids_hbm, tids_hbm, tth_hbm, ttt_hbm, he_out, te_out,
                 ids_v, mid_v, mpos_v, buf2, stag2, sem, sem2):
    wid = lax.axis_index("s") * NC + lax.axis_index("c")

    h_lo = wid * (h_tc_pw * TCW)
    h_hi = jnp.minimum(h_lo + h_tc_pw * TCW, VH)
    _extract_phase(hids_hbm, tth_hbm, he_out, ids_v, mid_v, mpos_v, buf2,
                   stag2, sem, sem2, h_lo, h_hi, B)
    plsc.subcore_barrier()
    t_lo = wid * (t_tc_pw * TCW)
    t_hi = jnp.minimum(t_lo + t_tc_pw * TCW, VT)
    _extract_phase(tids_hbm, ttt_hbm, te_out, ids_v, mid_v, mpos_v, buf2,
                   stag2, sem, sem2, t_lo, t_hi, B)


def _sc_gather(house_ids, time_ids, house_table, time_table):
    B = house_ids.shape[0]
    VH = house_table.shape[0]
    VT = time_table.shape[0]
    h_tc = -(-VH // TCW)
    t_tc = -(-VT // TCW)
    h_tc_pw = -(-h_tc // NW)
    t_tc_pw = -(-t_tc // NW)
    body = functools.partial(_gather_body, B, VH, VT, h_tc_pw, t_tc_pw)
    return pl.kernel(
        body,
        out_type=(
            jax.ShapeDtypeStruct((B + L, 2 * HOUSE_DIM), jnp.float32),
            jax.ShapeDtypeStruct((B + L, 2 * TIME_DIM), jnp.float32),
        ),
        mesh=plsc.VectorSubcoreMesh(
            core_axis_name="c", subcore_axis_name="s",
            num_cores=NC, num_subcores=NS),
        scratch_types=[
            pltpu.VMEM((B,), jnp.int32),
            pltpu.VMEM((B + L + 1,), jnp.int32),
            pltpu.VMEM((B + L + 1,), jnp.int32),
            pltpu.VMEM((2, HOUSE_DIM, TCW), jnp.float32),
            pltpu.VMEM((2, L, 2 * HOUSE_DIM), jnp.float32),
            pltpu.SemaphoreType.DMA,
            pltpu.SemaphoreType.DMA,
        ],
        compiler_params=pltpu.CompilerParams(needs_layout_passes=False),
    )(house_ids, time_ids, house_table.T, time_table.T)


def _mlp_body(he_ref, te_ref, w1_ref, b1_ref, w2_ref, b2_ref, w3_ref, b3_ref,
              out_ref):
    h = jnp.dot(he_ref[:, :HOUSE_DIM], w1_ref[:HOUSE_DIM, :],
                preferred_element_type=jnp.float32)
    h += jnp.dot(te_ref[:, :TIME_DIM], w1_ref[HOUSE_DIM:, :],
                 preferred_element_type=jnp.float32)
    h = jnp.maximum(h + b1_ref[...], 0.0)
    h = jnp.maximum(
        jnp.dot(h, w2_ref[...], preferred_element_type=jnp.float32)
        + b2_ref[...], 0.0)
    out_ref[...] = (jnp.dot(h, w3_ref[...], preferred_element_type=jnp.float32)
                    + b3_ref[...])


def _tc_mlp(he, te, W1, b1, W2, b2, W3, b3):
    B = he.shape[0] - L
    d1 = W1.shape[1]
    d2 = W2.shape[1]
    grid = (B // MLP_BLK,)
    full = lambda shape: pl.BlockSpec(shape, lambda i: (0, 0))
    return pl.pallas_call(
        _mlp_body,
        grid=grid,
        in_specs=[
            pl.BlockSpec((MLP_BLK, 2 * HOUSE_DIM), lambda i: (i, 0)),
            pl.BlockSpec((MLP_BLK, 2 * TIME_DIM), lambda i: (i, 0)),
            full(W1.shape),
            full((1, d1)),
            full(W2.shape),
            full((1, d2)),
            full(W3.shape),
            full((1, 1)),
        ],
        out_specs=pl.BlockSpec((MLP_BLK, 1), lambda i: (i, 0)),
        out_shape=jax.ShapeDtypeStruct((B, 1), jnp.float32),
    )(he, te, W1, b1.reshape(1, d1), W2, b2.reshape(1, d2), W3,
      b3.reshape(1, 1))


def kernel(house_ids, time_ids, house_table, time_table, W1, b1, W2, b2, W3,
           b3):
    he, te = _sc_gather(house_ids.astype(jnp.int32),
                        time_ids.astype(jnp.int32),
                        house_table, time_table)
    return _tc_mlp(he, te, W1, b1, W2, b2, W3, b3)


# speedup vs baseline: 1.0093x; 1.0093x over previous
import functools

import jax
import jax.numpy as jnp
from jax import lax
from jax.experimental import pallas as pl
from jax.experimental.pallas import tpu as pltpu
from jax.experimental.pallas import tpu_sc as plsc

HOUSE_DIM = 64
TIME_DIM = 64
NC, NS, L = 2, 16, 16
NW = NC * NS
TCW = 128
MLP_BLK = 2048
NBUF = 8


def _bcast(x, dtype=jnp.int32):
    return jnp.full((L,), x, dtype)


def _extract_phase(ids_hbm, tt_hbm, out_hbm, ids_v, mid_v, mpos_v, buf2,
                   stag2, sem, sem2, lo, hi, B):
    pltpu.sync_copy(ids_hbm, ids_v)

    def scan_step(k, n):
        v = ids_v[pl.ds(k * L, L)]
        m = (v >= _bcast(lo)) & (v < _bcast(hi))
        pos = lax.iota(jnp.int32, L) + _bcast(k * L)
        mi = m.astype(jnp.int32)
        cum = plsc.cumsum(mi)
        tgt = jnp.where(m, _bcast(n - 1) + cum, _bcast(B + L))
        plsc.store_scatter(mid_v, [tgt], v)
        plsc.store_scatter(mpos_v, [tgt], pos)
        return n + jnp.sum(mi)

    n = lax.fori_loop(0, B // L, scan_step, jnp.int32(0))

    nch = (hi - lo + TCW - 1) // TCW
    ngrp = (n + L - 1) // L

    def start_chunk(c, r):
        off = lo + c * TCW
        return pltpu.async_copy(tt_hbm.at[:, pl.ds(off, TCW)], buf2.at[r],
                                sem)

    for c0 in range(NBUF):
        @pl.when(c0 < nch)
        def _prime(c0=c0):
            start_chunk(c0, c0)

    def chunk_step(c, it):
        r = c % NBUF
        off = lo + c * TCW
        pltpu.make_async_copy(tt_hbm.at[:, pl.ds(off, TCW)], buf2.at[r],
                              sem).wait()

        @pl.when(c + NBUF < nch)
        def _prefetch():
            start_chunk(c + NBUF, r)

        def grp_step(g, it):
            vid = mid_v[pl.ds(g * L, L)]
            vpos = mpos_v[pl.ds(g * L, L)]
            valid = (lax.iota(jnp.int32, L) + _bcast(g * L)) < _bcast(n)
            m = valid & (vid >= _bcast(off)) & (vid < _bcast(off + TCW))
            nact = jnp.sum(m.astype(jnp.int32))

            @pl.when(nact > 0)
            def _work():
                s = it % 2
                @pl.when(it >= 2)
                def _drain():
                    pltpu.make_async_copy(
                        stag2.at[s], out_hbm.at[lax.iota(jnp.int32, L)],
                        sem2).wait()

                col = jnp.where(m, vid - _bcast(off), _bcast(0))
                for f in range(HOUSE_DIM):
                    vals = plsc.load_gather(buf2.at[r], [_bcast(f), col])
                    plsc.store_scatter(
                        stag2.at[s],
                        [lax.iota(jnp.int32, L), _bcast(f)], vals)
                tgt = jnp.where(m, vpos, _bcast(B) + lax.iota(jnp.int32, L))
                pltpu.async_copy(stag2.at[s], out_hbm.at[tgt], sem2)

            return it + jnp.where(nact > 0, 1, 0)

        return lax.fori_loop(0, ngrp, grp_step, it)

    it = lax.fori_loop(0, nch, chunk_step, jnp.int32(0))

    @pl.when(it >= 2)
    def _d2():
        pltpu.make_async_copy(stag2.at[0], out_hbm.at[lax.iota(jnp.int32, L)],
                              sem2).wait()

    @pl.when(it >= 1)
    def _d1():
        pltpu.make_async_copy(stag2.at[0], out_hbm.at[lax.iota(jnp.int32, L)],
                              sem2).wait()


def _gather_body(B, VH, VT, h_tc_pw, t_tc_pw,
                 hids_hbm, tids_hbm, tth_hbm, ttt_hbm, he_out, te_out,
                 ids_v, mid_v, mpos_v, buf2, stag2, sem, sem2):
    wid = lax.axis_index("s") * NC + lax.axis_index("c")

    h_lo = wid * (h_tc_pw * TCW)
    h_hi = jnp.minimum(h_lo + h_tc_pw * TCW, VH)
    _extract_phase(hids_hbm, tth_hbm, he_out, ids_v, mid_v, mpos_v, buf2,
                   stag2, sem, sem2, h_lo, h_hi, B)
    plsc.subcore_barrier()
    t_lo = wid * (t_tc_pw * TCW)
    t_hi = jnp.minimum(t_lo + t_tc_pw * TCW, VT)
    _extract_phase(tids_hbm, ttt_hbm, te_out, ids_v, mid_v, mpos_v, buf2,
                   stag2, sem, sem2, t_lo, t_hi, B)


def _sc_gather(house_ids, time_ids, house_table, time_table):
    B = house_ids.shape[0]
    VH = house_table.shape[0]
    VT = time_table.shape[0]
    h_tc = -(-VH // TCW)
    t_tc = -(-VT // TCW)
    h_tc_pw = -(-h_tc // NW)
    t_tc_pw = -(-t_tc // NW)
    body = functools.partial(_gather_body, B, VH, VT, h_tc_pw, t_tc_pw)
    return pl.kernel(
        body,
        out_type=(
            jax.ShapeDtypeStruct((B + L, 2 * HOUSE_DIM), jnp.float32),
            jax.ShapeDtypeStruct((B + L, 2 * TIME_DIM), jnp.float32),
        ),
        mesh=plsc.VectorSubcoreMesh(
            core_axis_name="c", subcore_axis_name="s",
            num_cores=NC, num_subcores=NS),
        scratch_types=[
            pltpu.VMEM((B,), jnp.int32),
            pltpu.VMEM((B + L + 1,), jnp.int32),
            pltpu.VMEM((B + L + 1,), jnp.int32),
            pltpu.VMEM((NBUF, HOUSE_DIM, TCW), jnp.float32),
            pltpu.VMEM((2, L, 2 * HOUSE_DIM), jnp.float32),
            pltpu.SemaphoreType.DMA,
            pltpu.SemaphoreType.DMA,
        ],
        compiler_params=pltpu.CompilerParams(needs_layout_passes=False),
    )(house_ids, time_ids, house_table.T, time_table.T)


def _mlp_body(he_ref, te_ref, w1_ref, b1_ref, w2_ref, b2_ref, w3_ref, b3_ref,
              out_ref):
    h = jnp.dot(he_ref[:, :HOUSE_DIM], w1_ref[:HOUSE_DIM, :],
                preferred_element_type=jnp.float32)
    h += jnp.dot(te_ref[:, :TIME_DIM], w1_ref[HOUSE_DIM:, :],
                 preferred_element_type=jnp.float32)
    h = jnp.maximum(h + b1_ref[...], 0.0)
    h = jnp.maximum(
        jnp.dot(h, w2_ref[...], preferred_element_type=jnp.float32)
        + b2_ref[...], 0.0)
    out_ref[...] = (jnp.dot(h, w3_ref[...], preferred_element_type=jnp.float32)
                    + b3_ref[...])


def _tc_mlp(he, te, W1, b1, W2, b2, W3, b3):
    B = he.shape[0] - L
    d1 = W1.shape[1]
    d2 = W2.shape[1]
    grid = (B // MLP_BLK,)
    full = lambda shape: pl.BlockSpec(shape, lambda i: (0, 0))
    return pl.pallas_call(
        _mlp_body,
        grid=grid,
        in_specs=[
            pl.BlockSpec((MLP_BLK, 2 * HOUSE_DIM), lambda i: (i, 0)),
            pl.BlockSpec((MLP_BLK, 2 * TIME_DIM), lambda i: (i, 0)),
            full(W1.shape),
            full((1, d1)),
            full(W2.shape),
            full((1, d2)),
            full(W3.shape),
            full((1, 1)),
        ],
        out_specs=pl.BlockSpec((MLP_BLK, 1), lambda i: (i, 0)),
        out_shape=jax.ShapeDtypeStruct((B, 1), jnp.float32),
    )(he, te, W1, b1.reshape(1, d1), W2, b2.reshape(1, d2), W3,
      b3.reshape(1, 1))


def kernel(house_ids, time_ids, house_table, time_table, W1, b1, W2, b2, W3,
           b3):
    he, te = _sc_gather(house_ids.astype(jnp.int32),
                        time_ids.astype(jnp.int32),
                        house_table, time_table)
    return _tc_mlp(he, te, W1, b1, W2, b2, W3, b3)


# speedup vs baseline: 1.2569x; 1.2454x over previous
import functools

import jax
import jax.numpy as jnp
from jax import lax
from jax.experimental import pallas as pl
from jax.experimental.pallas import tpu as pltpu
from jax.experimental.pallas import tpu_sc as plsc

HOUSE_DIM = 64
TIME_DIM = 64
NC, NS, L = 2, 16, 16
NW = NC * NS
TCW = 128
CW = 512
NBUF = 2
MLP_BLK = 2048


def _bcast(x, dtype=jnp.int32):
    return jnp.full((L,), x, dtype)


def _extract_phase(ids_hbm, tt_hbm, out_hbm, ids_v, mid_v, mpos_v, wbuf,
                   stag2, sem, sem2, lo, pad_hi, log_hi, B):
    nch = (pad_hi - lo + CW - 1) // CW

    def window_dmas(w, r, do_issue):
        off = lo + w * CW
        rem = pad_hi - off

        def one(width):
            for tr in range(HOUSE_DIM // 8):
                src = tt_hbm.at[pl.ds(tr * 8, 8), pl.ds(off, width)]
                dst = wbuf.at[r, pl.ds(tr * 8, 8), pl.ds(0, width)]
                if do_issue:
                    pltpu.async_copy(src, dst, sem)
                else:
                    pltpu.make_async_copy(src, dst, sem).wait()

        @pl.when(rem >= CW)
        def _full():
            one(CW)

        @pl.when(rem == 256)
        def _half():
            one(256)

        @pl.when(rem == 128)
        def _quarter():
            one(128)

    for w0 in range(NBUF):
        @pl.when(w0 < nch)
        def _prime(w0=w0):
            window_dmas(w0, w0, True)

    pltpu.sync_copy(ids_hbm, ids_v)

    def scan_step(k, n):
        v = ids_v[pl.ds(k * L, L)]
        m = (v >= _bcast(lo)) & (v < _bcast(log_hi))
        pos = lax.iota(jnp.int32, L) + _bcast(k * L)
        mi = m.astype(jnp.int32)
        cum = plsc.cumsum(mi)
        tgt = jnp.where(m, _bcast(n - 1) + cum, _bcast(B + L))
        plsc.store_scatter(mid_v, [tgt], v)
        plsc.store_scatter(mpos_v, [tgt], pos)
        return n + jnp.sum(mi)

    n = lax.fori_loop(0, B // L, scan_step, jnp.int32(0))
    ngrp = (n + L - 1) // L

    def chunk_step(w, it):
        r = w % NBUF
        off = lo + w * CW
        window_dmas(w, r, False)

        def grp_step(g, it):
            vid = mid_v[pl.ds(g * L, L)]
            vpos = mpos_v[pl.ds(g * L, L)]
            valid = (lax.iota(jnp.int32, L) + _bcast(g * L)) < _bcast(n)
            m = valid & (vid >= _bcast(off)) & (vid < _bcast(off + CW))
            nact = jnp.sum(m.astype(jnp.int32))

            @pl.when(nact > 0)
            def _work():
                s = it % 2
                @pl.when(it >= 2)
                def _drain():
                    pltpu.make_async_copy(
                        stag2.at[s], out_hbm.at[lax.iota(jnp.int32, L)],
                        sem2).wait()

                col = jnp.where(m, vid - _bcast(off), _bcast(0))
                for f in range(HOUSE_DIM):
                    vals = plsc.load_gather(wbuf.at[r], [_bcast(f), col])
                    plsc.store_scatter(
                        stag2.at[s],
                        [lax.iota(jnp.int32, L), _bcast(f)], vals)
                tgt = jnp.where(m, vpos, _bcast(B) + lax.iota(jnp.int32, L))
                pltpu.async_copy(stag2.at[s], out_hbm.at[tgt], sem2)

            return it + jnp.where(nact > 0, 1, 0)

        it = lax.fori_loop(0, ngrp, grp_step, it)

        @pl.when(w + NBUF < nch)
        def _prefetch():
            window_dmas(w + NBUF, r, True)

        return it

    it = lax.fori_loop(0, nch, chunk_step, jnp.int32(0))

    @pl.when(it >= 2)
    def _d2():
        pltpu.make_async_copy(stag2.at[0], out_hbm.at[lax.iota(jnp.int32, L)],
                              sem2).wait()

    @pl.when(it >= 1)
    def _d1():
        pltpu.make_async_copy(stag2.at[0], out_hbm.at[lax.iota(jnp.int32, L)],
                              sem2).wait()


def _gather_body(B, VH, VT, h_cw_pw, t_cw_pw, h_pad, t_pad,
                 hids_hbm, tids_hbm, tth_hbm, ttt_hbm, he_out, te_out,
                 ids_v, mid_v, mpos_v, wbuf, stag2, sem, sem2):
    wid = lax.axis_index("s") * NC + lax.axis_index("c")

    h_lo = wid * (h_cw_pw * CW)
    h_pad_hi = jnp.minimum(h_lo + h_cw_pw * CW, h_pad)
    h_log_hi = jnp.minimum(h_pad_hi, VH)
    _extract_phase(hids_hbm, tth_hbm, he_out, ids_v, mid_v, mpos_v, wbuf,
                   stag2, sem, sem2, h_lo, h_pad_hi, h_log_hi, B)
    plsc.subcore_barrier()
    t_lo = wid * (t_cw_pw * CW)
    t_pad_hi = jnp.minimum(t_lo + t_cw_pw * CW, t_pad)
    t_log_hi = jnp.minimum(t_pad_hi, VT)
    _extract_phase(tids_hbm, ttt_hbm, te_out, ids_v, mid_v, mpos_v, wbuf,
                   stag2, sem, sem2, t_lo, t_pad_hi, t_log_hi, B)


def _sc_gather(house_ids, time_ids, house_table, time_table):
    B = house_ids.shape[0]
    VH = house_table.shape[0]
    VT = time_table.shape[0]
    h_pad = -(-VH // TCW) * TCW
    t_pad = -(-VT // TCW) * TCW
    h_cw_pw = -(-h_pad // (NW * CW))
    t_cw_pw = -(-t_pad // (NW * CW))
    body = functools.partial(_gather_body, B, VH, VT, h_cw_pw, t_cw_pw,
                             h_pad, t_pad)
    return pl.kernel(
        body,
        out_type=(
            jax.ShapeDtypeStruct((B + L, 2 * HOUSE_DIM), jnp.float32),
            jax.ShapeDtypeStruct((B + L, 2 * TIME_DIM), jnp.float32),
        ),
        mesh=plsc.VectorSubcoreMesh(
            core_axis_name="c", subcore_axis_name="s",
            num_cores=NC, num_subcores=NS),
        scratch_types=[
            pltpu.VMEM((B,), jnp.int32),
            pltpu.VMEM((B + L + 1,), jnp.int32),
            pltpu.VMEM((B + L + 1,), jnp.int32),
            pltpu.VMEM((NBUF, HOUSE_DIM, CW), jnp.float32),
            pltpu.VMEM((2, L, 2 * HOUSE_DIM), jnp.float32),
            pltpu.SemaphoreType.DMA,
            pltpu.SemaphoreType.DMA,
        ],
        compiler_params=pltpu.CompilerParams(needs_layout_passes=False),
    )(house_ids, time_ids, house_table.T, time_table.T)


def _mlp_body(he_ref, te_ref, w1_ref, b1_ref, w2_ref, b2_ref, w3_ref, b3_ref,
              out_ref):
    h = jnp.dot(he_ref[:, :HOUSE_DIM], w1_ref[:HOUSE_DIM, :],
                preferred_element_type=jnp.float32)
    h += jnp.dot(te_ref[:, :TIME_DIM], w1_ref[HOUSE_DIM:, :],
                 preferred_element_type=jnp.float32)
    h = jnp.maximum(h + b1_ref[...], 0.0)
    h = jnp.maximum(
        jnp.dot(h, w2_ref[...], preferred_element_type=jnp.float32)
        + b2_ref[...], 0.0)
    out_ref[...] = (jnp.dot(h, w3_ref[...], preferred_element_type=jnp.float32)
                    + b3_ref[...])


def _tc_mlp(he, te, W1, b1, W2, b2, W3, b3):
    B = he.shape[0] - L
    d1 = W1.shape[1]
    d2 = W2.shape[1]
    grid = (B // MLP_BLK,)
    full = lambda shape: pl.BlockSpec(shape, lambda i: (0, 0))
    return pl.pallas_call(
        _mlp_body,
        grid=grid,
        in_specs=[
            pl.BlockSpec((MLP_BLK, 2 * HOUSE_DIM), lambda i: (i, 0)),
            pl.BlockSpec((MLP_BLK, 2 * TIME_DIM), lambda i: (i, 0)),
            full(W1.shape),
            full((1, d1)),
            full(W2.shape),
            full((1, d2)),
            full(W3.shape),
            full((1, 1)),
        ],
        out_specs=pl.BlockSpec((MLP_BLK, 1), lambda i: (i, 0)),
        out_shape=jax.ShapeDtypeStruct((B, 1), jnp.float32),
    )(he, te, W1, b1.reshape(1, d1), W2, b2.reshape(1, d2), W3,
      b3.reshape(1, 1))


def kernel(house_ids, time_ids, house_table, time_table, W1, b1, W2, b2, W3,
           b3):
    he, te = _sc_gather(house_ids.astype(jnp.int32),
                        time_ids.astype(jnp.int32),
                        house_table, time_table)
    return _tc_mlp(he, te, W1, b1, W2, b2, W3, b3)


# speedup vs baseline: 2.3468x; 1.8672x over previous
import functools

import jax
import jax.numpy as jnp
from jax import lax
from jax.experimental import pallas as pl
from jax.experimental.pallas import tpu as pltpu
from jax.experimental.pallas import tpu_sc as plsc

HOUSE_DIM = 64
TIME_DIM = 64
NC, NS, L = 2, 16, 16
NW = NC * NS
TCW = 128
CW = 512
NBUF = 2
DUMP = 2048
MLP_BLK = 2048


def _bcast(x, dtype=jnp.int32):
    return jnp.full((L,), x, dtype)


def _extract_phase(ids_hbm, tt_hbm, out_hbm, ids_v, mid_v, mpos_v, wbuf,
                   stag2, sem, sem2, lo, pad_hi, log_hi, B):
    nch = (pad_hi - lo + CW - 1) // CW

    def window_dmas(w, r, do_issue):
        off = lo + w * CW
        rem = pad_hi - off

        def one(width):
            for tr in range(HOUSE_DIM // 8):
                src = tt_hbm.at[pl.ds(tr * 8, 8), pl.ds(off, width)]
                dst = wbuf.at[r, pl.ds(tr * 8, 8), pl.ds(0, width)]
                if do_issue:
                    pltpu.async_copy(src, dst, sem)
                else:
                    pltpu.make_async_copy(src, dst, sem).wait()

        @pl.when(rem >= CW)
        def _full():
            one(CW)

        @pl.when(rem == 256)
        def _half():
            one(256)

        @pl.when(rem == 128)
        def _quarter():
            one(128)

    for w0 in range(NBUF):
        @pl.when(w0 < nch)
        def _prime(w0=w0):
            window_dmas(w0, w0, True)

    pltpu.sync_copy(ids_hbm, ids_v)

    def scan_step(k, n):
        v = ids_v[pl.ds(k * L, L)]
        m = (v >= _bcast(lo)) & (v < _bcast(log_hi))
        pos = lax.iota(jnp.int32, L) + _bcast(k * L)
        mi = m.astype(jnp.int32)
        cum = plsc.cumsum(mi)
        tgt = jnp.where(m, _bcast(n - 1) + cum, _bcast(B + L))
        plsc.store_scatter(mid_v, [tgt], v)
        plsc.store_scatter(mpos_v, [tgt], pos)
        return n + jnp.sum(mi)

    n = lax.fori_loop(0, B // L, scan_step, jnp.int32(0))
    ngrp = (n + L - 1) // L

    def chunk_step(w, it):
        r = w % NBUF
        off = lo + w * CW
        window_dmas(w, r, False)

        def grp_step(g, it):
            vid = mid_v[pl.ds(g * L, L)]
            vpos = mpos_v[pl.ds(g * L, L)]
            valid = (lax.iota(jnp.int32, L) + _bcast(g * L)) < _bcast(n)
            m = valid & (vid >= _bcast(off)) & (vid < _bcast(off + CW))
            nact = jnp.sum(m.astype(jnp.int32))

            @pl.when(nact > 0)
            def _work():
                s = it % 2
                @pl.when(it >= 2)
                def _drain():
                    pltpu.make_async_copy(
                        stag2.at[s], out_hbm.at[lax.iota(jnp.int32, L)],
                        sem2).wait()

                col = jnp.where(m, vid - _bcast(off), _bcast(0))
                for f in range(HOUSE_DIM):
                    vals = plsc.load_gather(wbuf.at[r], [_bcast(f), col])
                    plsc.store_scatter(
                        stag2.at[s],
                        [lax.iota(jnp.int32, L), _bcast(f)], vals)
                spread = (vid * 13 + lax.iota(jnp.int32, L)
                          ) & _bcast(DUMP - 1)
                tgt = jnp.where(m, vpos, _bcast(B) + spread)
                pltpu.async_copy(stag2.at[s], out_hbm.at[tgt], sem2)

            return it + jnp.where(nact > 0, 1, 0)

        it = lax.fori_loop(0, ngrp, grp_step, it)

        @pl.when(w + NBUF < nch)
        def _prefetch():
            window_dmas(w + NBUF, r, True)

        return it

    it = lax.fori_loop(0, nch, chunk_step, jnp.int32(0))

    @pl.when(it >= 2)
    def _d2():
        pltpu.make_async_copy(stag2.at[0], out_hbm.at[lax.iota(jnp.int32, L)],
                              sem2).wait()

    @pl.when(it >= 1)
    def _d1():
        pltpu.make_async_copy(stag2.at[0], out_hbm.at[lax.iota(jnp.int32, L)],
                              sem2).wait()


def _gather_body(B, VH, VT, h_cw_pw, t_cw_pw, h_pad, t_pad,
                 hids_hbm, tids_hbm, tth_hbm, ttt_hbm, he_out, te_out,
                 ids_v, mid_v, mpos_v, wbuf, stag2, sem, sem2):
    wid = lax.axis_index("s") * NC + lax.axis_index("c")

    h_lo = wid * (h_cw_pw * CW)
    h_pad_hi = jnp.minimum(h_lo + h_cw_pw * CW, h_pad)
    h_log_hi = jnp.minimum(h_pad_hi, VH)
    _extract_phase(hids_hbm, tth_hbm, he_out, ids_v, mid_v, mpos_v, wbuf,
                   stag2, sem, sem2, h_lo, h_pad_hi, h_log_hi, B)
    plsc.subcore_barrier()
    t_lo = wid * (t_cw_pw * CW)
    t_pad_hi = jnp.minimum(t_lo + t_cw_pw * CW, t_pad)
    t_log_hi = jnp.minimum(t_pad_hi, VT)
    _extract_phase(tids_hbm, ttt_hbm, te_out, ids_v, mid_v, mpos_v, wbuf,
                   stag2, sem, sem2, t_lo, t_pad_hi, t_log_hi, B)


def _sc_gather(house_ids, time_ids, house_table, time_table):
    B = house_ids.shape[0]
    VH = house_table.shape[0]
    VT = time_table.shape[0]
    h_pad = -(-VH // TCW) * TCW
    t_pad = -(-VT // TCW) * TCW
    h_cw_pw = -(-h_pad // (NW * CW))
    t_cw_pw = -(-t_pad // (NW * CW))
    body = functools.partial(_gather_body, B, VH, VT, h_cw_pw, t_cw_pw,
                             h_pad, t_pad)
    return pl.kernel(
        body,
        out_type=(
            jax.ShapeDtypeStruct((B + DUMP, 2 * HOUSE_DIM), jnp.float32),
            jax.ShapeDtypeStruct((B + DUMP, 2 * TIME_DIM), jnp.float32),
        ),
        mesh=plsc.VectorSubcoreMesh(
            core_axis_name="c", subcore_axis_name="s",
            num_cores=NC, num_subcores=NS),
        scratch_types=[
            pltpu.VMEM((B,), jnp.int32),
            pltpu.VMEM((B + L + 1,), jnp.int32),
            pltpu.VMEM((B + L + 1,), jnp.int32),
            pltpu.VMEM((NBUF, HOUSE_DIM, CW), jnp.float32),
            pltpu.VMEM((2, L, 2 * HOUSE_DIM), jnp.float32),
            pltpu.SemaphoreType.DMA,
            pltpu.SemaphoreType.DMA,
        ],
        compiler_params=pltpu.CompilerParams(needs_layout_passes=False),
    )(house_ids, time_ids, house_table.T, time_table.T)


def _mlp_body(he_ref, te_ref, w1_ref, b1_ref, w2_ref, b2_ref, w3_ref, b3_ref,
              out_ref):
    h = jnp.dot(he_ref[:, :HOUSE_DIM], w1_ref[:HOUSE_DIM, :],
                preferred_element_type=jnp.float32)
    h += jnp.dot(te_ref[:, :TIME_DIM], w1_ref[HOUSE_DIM:, :],
                 preferred_element_type=jnp.float32)
    h = jnp.maximum(h + b1_ref[...], 0.0)
    h = jnp.maximum(
        jnp.dot(h, w2_ref[...], preferred_element_type=jnp.float32)
        + b2_ref[...], 0.0)
    out_ref[...] = (jnp.dot(h, w3_ref[...], preferred_element_type=jnp.float32)
                    + b3_ref[...])


def _tc_mlp(he, te, W1, b1, W2, b2, W3, b3):
    B = he.shape[0] - DUMP
    d1 = W1.shape[1]
    d2 = W2.shape[1]
    grid = (B // MLP_BLK,)
    full = lambda shape: pl.BlockSpec(shape, lambda i: (0, 0))
    return pl.pallas_call(
        _mlp_body,
        grid=grid,
        in_specs=[
            pl.BlockSpec((MLP_BLK, 2 * HOUSE_DIM), lambda i: (i, 0)),
            pl.BlockSpec((MLP_BLK, 2 * TIME_DIM), lambda i: (i, 0)),
            full(W1.shape),
            full((1, d1)),
            full(W2.shape),
            full((1, d2)),
            full(W3.shape),
            full((1, 1)),
        ],
        out_specs=pl.BlockSpec((MLP_BLK, 1), lambda i: (i, 0)),
        out_shape=jax.ShapeDtypeStruct((B, 1), jnp.float32),
    )(he, te, W1, b1.reshape(1, d1), W2, b2.reshape(1, d2), W3,
      b3.reshape(1, 1))


def kernel(house_ids, time_ids, house_table, time_table, W1, b1, W2, b2, W3,
           b3):
    he, te = _sc_gather(house_ids.astype(jnp.int32),
                        time_ids.astype(jnp.int32),
                        house_table, time_table)
    return _tc_mlp(he, te, W1, b1, W2, b2, W3, b3)
